# initial kernel scaffold (unmeasured)
import jax
import jax.numpy as jnp
from jax import lax
from jax.experimental import pallas as pl
from jax.experimental.pallas import tpu as pltpu

N_DEV = 8
M, K, N = 4096, 4096, 8192
KS = K // N_DEV
BN = 256


def _ring_neighbors():
    my = lax.axis_index("i")
    right = lax.rem(my + 1, N_DEV)
    left = lax.rem(my + N_DEV - 1, N_DEV)
    return my, left, right


def _neighbor_barrier(left, right):
    barrier = pltpu.get_barrier_semaphore()
    for nbr in (left, right):
        pl.semaphore_signal(
            barrier, inc=1,
            device_id=(nbr,), device_id_type=pl.DeviceIdType.MESH,
        )
    pl.semaphore_wait(barrier, 2)


def _ring_hops(slots, send_sems, recv_sems, right):
    for h in range(N_DEV - 1):
        rdma = pltpu.make_async_remote_copy(
            src_ref=slots.at[h],
            dst_ref=slots.at[h + 1],
            send_sem=send_sems.at[h],
            recv_sem=recv_sems.at[h],
            device_id=(right,),
            device_id_type=pl.DeviceIdType.MESH,
        )
        rdma.start()
        rdma.wait()
        yield h, h + 1


def _gather_x_body(x_ref, xbf_ref, slots, send_sems, recv_sems):
    my, left, right = _ring_neighbors()
    _neighbor_barrier(left, right)

    slots[0] = x_ref[...]
    xbf_ref[:, pl.ds(my * KS, KS)] = x_ref[...].astype(jnp.bfloat16)

    for h, slot in _ring_hops(slots, send_sems, recv_sems, right):
        origin = lax.rem(my + N_DEV - (h + 1), N_DEV)
        xbf_ref[:, pl.ds(origin * KS, KS)] = slots[slot].astype(jnp.bfloat16)


def _gather_w_body(w_ref, wout_ref, slots, send_sems, recv_sems, cp_sem):
    my, left, right = _ring_neighbors()
    _neighbor_barrier(left, right)

    slots[0] = w_ref[...]
    cp = pltpu.make_async_copy(
        slots.at[0], wout_ref.at[pl.ds(my * KS, KS), :], cp_sem
    )
    cp.start()
    cp.wait()

    for h, slot in _ring_hops(slots, send_sems, recv_sems, right):
        origin = lax.rem(my + N_DEV - (h + 1), N_DEV)
        cp = pltpu.make_async_copy(
            slots.at[slot], wout_ref.at[pl.ds(origin * KS, KS), :], cp_sem
        )
        cp.start()
        cp.wait()


def _gemm_body(xbf_ref, w_ref, s_ref, o_ref):
    wbf = w_ref[...].astype(jnp.bfloat16)
    acc = jnp.dot(xbf_ref[...], wbf, preferred_element_type=jnp.float32)
    o_ref[...] = jnp.maximum(acc * s_ref[0, 0], 0.0)


def kernel(x, w_mat, scale_x, scale_w):
    xbf = pl.pallas_call(
        _gather_x_body,
        out_shape=jax.ShapeDtypeStruct((M, K), jnp.bfloat16),
        in_specs=[pl.BlockSpec(memory_space=pltpu.VMEM)],
        out_specs=pl.BlockSpec(memory_space=pltpu.VMEM),
        scratch_shapes=[
            pltpu.VMEM((N_DEV, M, KS), jnp.int8),
            pltpu.SemaphoreType.DMA((N_DEV - 1,)),
            pltpu.SemaphoreType.DMA((N_DEV - 1,)),
        ],
        compiler_params=pltpu.CompilerParams(collective_id=0),
    )(x)

    w_full = pl.pallas_call(
        _gather_w_body,
        out_shape=jax.ShapeDtypeStruct((K, N), jnp.int8),
        in_specs=[pl.BlockSpec(memory_space=pltpu.VMEM)],
        out_specs=pl.BlockSpec(memory_space=pltpu.ANY),
        scratch_shapes=[
            pltpu.VMEM((N_DEV, KS, N), jnp.int8),
            pltpu.SemaphoreType.DMA((N_DEV - 1,)),
            pltpu.SemaphoreType.DMA((N_DEV - 1,)),
            pltpu.SemaphoreType.DMA,
        ],
        compiler_params=pltpu.CompilerParams(collective_id=1),
    )(w_mat)

    s = (scale_x * scale_w).astype(jnp.float32).reshape(1, 1)

    return pl.pallas_call(
        _gemm_body,
        grid=(N // BN,),
        in_specs=[
            pl.BlockSpec((M, K), lambda n: (0, 0)),
            pl.BlockSpec((K, BN), lambda n: (0, n)),
            pl.BlockSpec((1, 1), lambda n: (0, 0), memory_space=pltpu.SMEM),
        ],
        out_specs=pl.BlockSpec((M, BN), lambda n: (0, n)),
        out_shape=jax.ShapeDtypeStruct((M, N), jnp.float32),
    )(xbf, w_full, s)


# baseline (device time: 843050 ns/iter reference)
import functools

import jax
import jax.numpy as jnp
from jax import lax
from jax.experimental import pallas as pl
from jax.experimental.pallas import tpu as pltpu

N_DEV = 8
M, K, N = 4096, 4096, 8192
KS = K // N_DEV
BM = 1024
BN = 512


def _ring_neighbors():
    my = lax.axis_index("i")
    right = lax.rem(my + 1, N_DEV)
    left = lax.rem(my + N_DEV - 1, N_DEV)
    return my, left, right


def _neighbor_barrier(left, right):
    barrier = pltpu.get_barrier_semaphore()
    for nbr in (left, right):
        pl.semaphore_signal(
            barrier, inc=1,
            device_id=(nbr,), device_id_type=pl.DeviceIdType.MESH,
        )
    pl.semaphore_wait(barrier, 2)


def _exit_barrier(left, right):

    @functools.partial(pl.run_scoped, sem=pltpu.SemaphoreType.REGULAR)
    def _(sem):
        for nbr in (left, right):
            pl.semaphore_signal(
                sem, inc=1,
                device_id=(nbr,), device_id_type=pl.DeviceIdType.MESH,
            )
        pl.semaphore_wait(sem, 2)


def _ring_hops(slots, send_sems, recv_sems, right):
    for h in range(N_DEV - 1):
        rdma = pltpu.make_async_remote_copy(
            src_ref=slots.at[h],
            dst_ref=slots.at[h + 1],
            send_sem=send_sems.at[h],
            recv_sem=recv_sems.at[h],
            device_id=(right,),
            device_id_type=pl.DeviceIdType.MESH,
        )
        rdma.start()
        rdma.wait()
        yield h, h + 1


def _gather_x_body(x_ref, xbf_ref, slots, send_sems, recv_sems):
    my, left, right = _ring_neighbors()
    _neighbor_barrier(left, right)

    slots[0] = x_ref[...]
    xbf_ref[:, pl.ds(my * KS, KS)] = x_ref[...].astype(jnp.bfloat16)

    for h, slot in _ring_hops(slots, send_sems, recv_sems, right):
        origin = lax.rem(my + N_DEV - (h + 1), N_DEV)
        xbf_ref[:, pl.ds(origin * KS, KS)] = slots[slot].astype(jnp.bfloat16)

    _exit_barrier(left, right)


def _gather_w_body(w_ref, wout_ref, slots, send_sems, recv_sems, cp_sem):
    my, left, right = _ring_neighbors()
    _neighbor_barrier(left, right)

    slots[0] = w_ref[...]
    cp = pltpu.make_async_copy(
        slots.at[0], wout_ref.at[pl.ds(my * KS, KS), :], cp_sem
    )
    cp.start()
    cp.wait()

    for h, slot in _ring_hops(slots, send_sems, recv_sems, right):
        origin = lax.rem(my + N_DEV - (h + 1), N_DEV)
        cp = pltpu.make_async_copy(
            slots.at[slot], wout_ref.at[pl.ds(origin * KS, KS), :], cp_sem
        )
        cp.start()
        cp.wait()

    _exit_barrier(left, right)


def _gemm_body(xbf_ref, w_ref, s_ref, o_ref):
    wbf = w_ref[...].astype(jnp.bfloat16)
    acc = jnp.dot(xbf_ref[...], wbf, preferred_element_type=jnp.float32)
    o_ref[...] = jnp.maximum(acc * s_ref[0, 0], 0.0)


def _call_gather_x(x):
    return pl.pallas_call(
        _gather_x_body,
        out_shape=jax.ShapeDtypeStruct((M, K), jnp.bfloat16),
        in_specs=[pl.BlockSpec(memory_space=pltpu.VMEM)],
        out_specs=pl.BlockSpec(memory_space=pltpu.VMEM),
        scratch_shapes=[
            pltpu.VMEM((N_DEV, M, KS), jnp.int8),
            pltpu.SemaphoreType.DMA((N_DEV - 1,)),
            pltpu.SemaphoreType.DMA((N_DEV - 1,)),
        ],
        compiler_params=pltpu.CompilerParams(
            collective_id=0, vmem_limit_bytes=60 * 1024 * 1024
        ),
    )(x)


def _call_gather_w(w_mat):
    return pl.pallas_call(
        _gather_w_body,
        out_shape=jax.ShapeDtypeStruct((K, N), jnp.int8),
        in_specs=[pl.BlockSpec(memory_space=pltpu.VMEM)],
        out_specs=pl.BlockSpec(memory_space=pl.ANY),
        scratch_shapes=[
            pltpu.VMEM((N_DEV, KS, N), jnp.int8),
            pltpu.SemaphoreType.DMA((N_DEV - 1,)),
            pltpu.SemaphoreType.DMA((N_DEV - 1,)),
            pltpu.SemaphoreType.DMA,
        ],
        compiler_params=pltpu.CompilerParams(
            collective_id=1, vmem_limit_bytes=60 * 1024 * 1024
        ),
    )(w_mat)


def kernel(x, w_mat, scale_x, scale_w):
    xbf = _call_gather_x(x)
    w_full = _call_gather_w(w_mat)
    s = (scale_x * scale_w).astype(jnp.float32).reshape(1, 1)

    return pl.pallas_call(
        _gemm_body,
        grid=(M // BM, N // BN),
        in_specs=[
            pl.BlockSpec((BM, K), lambda m, n: (m, 0)),
            pl.BlockSpec((K, BN), lambda m, n: (0, n)),
            pl.BlockSpec((1, 1), lambda m, n: (0, 0), memory_space=pltpu.SMEM),
        ],
        out_specs=pl.BlockSpec((BM, BN), lambda m, n: (m, n)),
        out_shape=jax.ShapeDtypeStruct((M, N), jnp.float32),
        compiler_params=pltpu.CompilerParams(
            vmem_limit_bytes=60 * 1024 * 1024
        ),
    )(xbf, w_full, s)


# device time: 561482 ns/iter; 1.5015x vs baseline; 1.5015x over previous
import functools

import jax
import jax.numpy as jnp
from jax import lax
from jax.experimental import pallas as pl
from jax.experimental.pallas import tpu as pltpu

N_DEV = 8
M, K, N = 4096, 4096, 8192
KS = K // N_DEV
BM = 1024
BN = 1024


def _ring_neighbors():
    my = lax.axis_index("i")
    right = lax.rem(my + 1, N_DEV)
    left = lax.rem(my + N_DEV - 1, N_DEV)
    return my, left, right


def _neighbor_barrier(left, right):
    barrier = pltpu.get_barrier_semaphore()
    for nbr in (left, right):
        pl.semaphore_signal(
            barrier, inc=1,
            device_id=(nbr,), device_id_type=pl.DeviceIdType.MESH,
        )
    pl.semaphore_wait(barrier, 2)


def _exit_barrier(left, right):

    @functools.partial(pl.run_scoped, sem=pltpu.SemaphoreType.REGULAR)
    def _(sem):
        for nbr in (left, right):
            pl.semaphore_signal(
                sem, inc=1,
                device_id=(nbr,), device_id_type=pl.DeviceIdType.MESH,
            )
        pl.semaphore_wait(sem, 2)


HS = KS // 2


def _gather_xw_body(x_ref, w_ref, xbf_ref, wbf_ref,
                    xs_f, xs_b, ws_f, ws_b, xstg_f, xstg_b, wstg,
                    xsend_f, xrecv_f, xsend_b, xrecv_b,
                    wsend_f, wrecv_f, wsend_b, wrecv_b,
                    cpxf_sem, cpxb_sem, cpw_sem):
    my, left, right = _ring_neighbors()
    _neighbor_barrier(left, right)

    def mk4(h):
        def rc(src, slots, send, recv, nbr):
            return pltpu.make_async_remote_copy(
                src_ref=src, dst_ref=slots.at[h],
                send_sem=send.at[h], recv_sem=recv.at[h],
                device_id=(nbr,), device_id_type=pl.DeviceIdType.MESH,
            )
        if h == 0:
            srcs = (x_ref.at[:, pl.ds(0, HS)], x_ref.at[:, pl.ds(HS, HS)],
                    w_ref.at[pl.ds(0, HS), :], w_ref.at[pl.ds(HS, HS), :])
        else:
            srcs = (xs_f.at[h - 1], xs_b.at[h - 1],
                    ws_f.at[h - 1], ws_b.at[h - 1])
        return (rc(srcs[0], xs_f, xsend_f, xrecv_f, right),
                rc(srcs[1], xs_b, xsend_b, xrecv_b, left),
                rc(srcs[2], ws_f, wsend_f, wrecv_f, right),
                rc(srcs[3], ws_b, wsend_b, wrecv_b, left))

    cpxf = cpxb = cpw = None

    def _process(xsrc_f, xsrc_b, wsrc_f, wsrc_b, origin_f, origin_b):
        nonlocal cpxf, cpxb, cpw
        if cpxf is not None:
            cpxf.wait()
        xstg_f[...] = xsrc_f.astype(jnp.bfloat16)
        cpxf = pltpu.make_async_copy(
            xstg_f, xbf_ref.at[:, pl.ds(origin_f * KS, HS)], cpxf_sem)
        cpxf.start()
        if cpxb is not None:
            cpxb.wait()
        xstg_b[...] = xsrc_b.astype(jnp.bfloat16)
        cpxb = pltpu.make_async_copy(
            xstg_b, xbf_ref.at[:, pl.ds(origin_b * KS + HS, HS)], cpxb_sem)
        cpxb.start()
        if cpw is not None:
            cpw.wait()
        wstg[...] = wsrc_f.astype(jnp.bfloat16)
        cpw = pltpu.make_async_copy(
            wstg, wbf_ref.at[pl.ds(origin_f * KS, HS), :], cpw_sem)
        cpw.start()
        cpw.wait()
        wstg[...] = wsrc_b.astype(jnp.bfloat16)
        cpw = pltpu.make_async_copy(
            wstg, wbf_ref.at[pl.ds(origin_b * KS + HS, HS), :], cpw_sem)
        cpw.start()

    rdmas = mk4(0)
    for r in rdmas:
        r.start()
    _process(x_ref[:, :HS], x_ref[:, HS:], w_ref[:HS, :], w_ref[HS:, :],
             my, my)

    for h in range(N_DEV - 1):
        for r in rdmas:
            r.wait()
        if h + 1 < N_DEV - 1:
            rdmas = mk4(h + 1)
            for r in rdmas:
                r.start()
        of = lax.rem(my + N_DEV - (h + 1), N_DEV)
        ob = lax.rem(my + h + 1, N_DEV)
        _process(xs_f[h], xs_b[h], ws_f[h], ws_b[h], of, ob)

    cpxf.wait()
    cpxb.wait()
    cpw.wait()
    _exit_barrier(left, right)


def _call_gather_xw(x, w_mat):
    return pl.pallas_call(
        _gather_xw_body,
        out_shape=[
            jax.ShapeDtypeStruct((M, K), jnp.bfloat16),
            jax.ShapeDtypeStruct((K, N), jnp.bfloat16),
        ],
        in_specs=[pl.BlockSpec(memory_space=pltpu.VMEM),
                  pl.BlockSpec(memory_space=pltpu.VMEM)],
        out_specs=[pl.BlockSpec(memory_space=pl.ANY),
                   pl.BlockSpec(memory_space=pl.ANY)],
        scratch_shapes=[
            pltpu.VMEM((N_DEV - 1, M, HS), jnp.int8),
            pltpu.VMEM((N_DEV - 1, M, HS), jnp.int8),
            pltpu.VMEM((N_DEV - 1, HS, N), jnp.int8),
            pltpu.VMEM((N_DEV - 1, HS, N), jnp.int8),
            pltpu.VMEM((M, HS), jnp.bfloat16),
            pltpu.VMEM((M, HS), jnp.bfloat16),
            pltpu.VMEM((HS, N), jnp.bfloat16),
        ] + [pltpu.SemaphoreType.DMA((N_DEV - 1,))] * 8 + [
            pltpu.SemaphoreType.DMA,
            pltpu.SemaphoreType.DMA,
            pltpu.SemaphoreType.DMA,
        ],
        compiler_params=pltpu.CompilerParams(
            collective_id=0, vmem_limit_bytes=63 * 1024 * 1024
        ),
    )(x, w_mat)


def _gemm_body(xbf_ref, w_ref, s_ref, o_ref):
    acc = jnp.dot(xbf_ref[...], w_ref[...],
                  preferred_element_type=jnp.float32)
    o_ref[...] = jnp.maximum(acc * s_ref[0, 0], 0.0)


def _call_gemm(xbf, w_full, s):
    return pl.pallas_call(
        _gemm_body,
        grid=(N // BN, M // BM),
        in_specs=[
            pl.BlockSpec((BM, K), lambda n, m: (m, 0)),
            pl.BlockSpec((K, BN), lambda n, m: (0, n)),
            pl.BlockSpec((1, 1), lambda n, m: (0, 0), memory_space=pltpu.SMEM),
        ],
        out_specs=pl.BlockSpec((BM, BN), lambda n, m: (m, n)),
        out_shape=jax.ShapeDtypeStruct((M, N), jnp.float32),
        compiler_params=pltpu.CompilerParams(
            vmem_limit_bytes=60 * 1024 * 1024
        ),
    )(xbf, w_full, s)


def kernel(x, w_mat, scale_x, scale_w):
    xbf, w_full = _call_gather_xw(x, w_mat)
    s = (scale_x * scale_w).astype(jnp.float32).reshape(1, 1)
    return _call_gemm(xbf, w_full, s)
